# Initial kernel scaffold; baseline (speedup 1.0000x reference)
import jax
import jax.numpy as jnp
from jax.experimental import pallas as pl


def kernel(x, mask, constr):
    raise NotImplementedError("write your pallas kernel here")


# speedup vs baseline: 2.0880x; 2.0880x over previous
import functools

import jax
import jax.numpy as jnp
from jax import lax
from jax.experimental import pallas as pl
from jax.experimental.pallas import tpu as pltpu
from jax.experimental.pallas import tpu_sc as plsc

B = 16384
C = 1000
NC = 2
NS = 16
NW = NC * NS
ROWS_PER_W = B // NW
RBLK = 8
NBLK = ROWS_PER_W // RBLK
BLK_ELEMS = RBLK * C
BUF = BLK_ELEMS + 16
NFULL = C // 16
TAIL = C - NFULL * 16


def _body(x_hbm, mask_hbm, constr_hbm, out_hbm,
          xa, xb, ma, mb, ca, cb, out_v,
          sxa, sxb, sma, smb, sca, scb):
    wid = lax.axis_index("s") * NC + lax.axis_index("c")
    base_elem = wid * (ROWS_PER_W * C)

    lanes = lax.iota(jnp.int32, 16)
    tail_mask = lanes < TAIL

    bufs = ((xa, ma, ca), (xb, mb, cb))
    sems = ((sxa, sma, sca), (sxb, smb, scb))
    hbms = (x_hbm, mask_hbm, constr_hbm)

    def start(blk, slot):
        off = base_elem + blk * BLK_ELEMS
        for hbm, buf, sem in zip(hbms, bufs[slot], sems[slot]):
            pltpu.async_copy(hbm.at[pl.ds(off, BLK_ELEMS)],
                             buf.at[pl.ds(0, BLK_ELEMS)], sem)

    def wait(blk, slot):
        off = base_elem + blk * BLK_ELEMS
        for hbm, buf, sem in zip(hbms, bufs[slot], sems[slot]):
            pltpu.make_async_copy(hbm.at[pl.ds(off, BLK_ELEMS)],
                                  buf.at[pl.ds(0, BLK_ELEMS)], sem).wait()

    def compute(blk, slot):
        xv, mv, cv = bufs[slot]
        for r in range(RBLK):
            rbase = r * C

            def chunk(c, acc):
                num, den = acc
                idx = lanes + (rbase + c * 16)
                xs = plsc.load_gather(xv, [idx])
                mk = plsc.load_gather(mv, [idx])
                g = plsc.load_gather(cv, [mk + rbase])
                e = jnp.exp(xs)
                return (num + e * g, den + e)

            zeros = jnp.zeros((16,), jnp.float32)
            num, den = lax.fori_loop(0, NFULL, chunk, (zeros, zeros))

            idx = lanes + (rbase + NFULL * 16)
            xs = plsc.load_gather(xv, [idx])
            mk = plsc.load_gather(mv, [idx])
            mk = jnp.where(tail_mask, mk, 0)
            g = plsc.load_gather(cv, [mk + rbase])
            e = jnp.where(tail_mask, jnp.exp(xs), 0.0)
            num = num + e * g
            den = den + e

            num_bc = jnp.full((16,), jnp.sum(num), jnp.float32)
            den_bc = jnp.full((16,), jnp.sum(den), jnp.float32)
            res_v = jnp.clip(num_bc / den_bc, 0.0, 1.0)
            row = blk * RBLK + r
            plsc.store_scatter(out_v,
                               [jnp.full((16,), row, jnp.int32)],
                               res_v,
                               mask=lanes == 0)

    start(0, 0)
    start(1, 1)

    def blkpair(i, carry):
        b0 = i * 2
        wait(b0, 0)
        compute(b0, 0)

        @pl.when(b0 + 2 < NBLK)
        def _():
            start(b0 + 2, 0)

        b1 = b0 + 1
        wait(b1, 1)
        compute(b1, 1)

        @pl.when(b1 + 2 < NBLK)
        def _():
            start(b1 + 2, 1)

        return carry

    lax.fori_loop(0, NBLK // 2, blkpair, 0)

    pltpu.sync_copy(out_v, out_hbm.at[pl.ds(wid * ROWS_PER_W, ROWS_PER_W)])


_rel_in_set = functools.partial(
    pl.kernel,
    out_type=jax.ShapeDtypeStruct((B,), jnp.float32),
    mesh=plsc.VectorSubcoreMesh(core_axis_name="c", subcore_axis_name="s"),
    scratch_types=[
        pltpu.VMEM((BUF,), jnp.float32),
        pltpu.VMEM((BUF,), jnp.float32),
        pltpu.VMEM((BUF,), jnp.int32),
        pltpu.VMEM((BUF,), jnp.int32),
        pltpu.VMEM((BUF,), jnp.float32),
        pltpu.VMEM((BUF,), jnp.float32),
        pltpu.VMEM((ROWS_PER_W,), jnp.float32),
        pltpu.SemaphoreType.DMA,
        pltpu.SemaphoreType.DMA,
        pltpu.SemaphoreType.DMA,
        pltpu.SemaphoreType.DMA,
        pltpu.SemaphoreType.DMA,
        pltpu.SemaphoreType.DMA,
    ],
    compiler_params=pltpu.CompilerParams(
        use_tc_tiling_on_sc=False,
        needs_layout_passes=False,
    ),
)(_body)


def kernel(x, mask, constr):
    return _rel_in_set(x.reshape(-1), mask.reshape(-1), constr.reshape(-1))
